# Initial kernel scaffold; baseline (speedup 1.0000x reference)
import numpy as np
import jax
import jax.numpy as jnp
from jax import lax
from jax.experimental import pallas as pl
from jax.experimental.pallas import tpu as pltpu

_THR = np.arange(0.5, 0.75, 0.05).astype(np.float32)
N_PRED = 5000
N_GT = 1000
N_GT_PAD = 1024
N_THR = len(_THR)


def _tc_body(px0, py0, px1, py1, gx0, gy0, gx1, gy1, thr, out_ref, mat_ref):
    g0x = gx0[...]
    g0y = gy0[...]
    g1x = gx1[...]
    g1y = gy1[...]
    garea = (g1x - g0x) * (g1y - g0y)

    def blk(b, _):
        r = b * 8
        p0x = px0[pl.ds(r, 8), :]
        p0y = py0[pl.ds(r, 8), :]
        p1x = px1[pl.ds(r, 8), :]
        p1y = py1[pl.ds(r, 8), :]
        parea = (p1x - p0x) * (p1y - p0y)
        ltx = jnp.maximum(p0x, g0x)
        lty = jnp.maximum(p0y, g0y)
        rbx = jnp.minimum(p1x, g1x)
        rby = jnp.minimum(p1y, g1y)
        whx = jnp.maximum(rbx - ltx, 0.0)
        why = jnp.maximum(rby - lty, 0.0)
        inter = whx * why
        union = parea + garea - inter
        mat_ref[pl.ds(r, 8), :] = inter / union
        return 0

    lax.fori_loop(0, N_PRED // 8, blk, 0)

    thrv = thr[...]
    col = lax.broadcasted_iota(jnp.int32, (8, N_GT_PAD), 1)

    def step(i, carry):
        matched, fp = carry
        row = mat_ref[pl.ds(i, 1), :]
        rowb = jnp.broadcast_to(row, (8, N_GT_PAD))
        masked = jnp.where(matched > 0.0, -1.0, rowb)
        mx = jnp.max(masked, axis=1, keepdims=True)
        ism = mx >= thrv
        idx = jnp.min(jnp.where(masked == mx, col, N_GT_PAD * 2), axis=1,
                      keepdims=True)
        matched = jnp.where(ism & (col == idx), 1.0, matched)
        fp = fp + jnp.where(ism, 0.0, 1.0)
        return matched, fp

    matched, fp = lax.fori_loop(
        0, N_PRED, step,
        (jnp.zeros((8, N_GT_PAD), jnp.float32), jnp.zeros((8, 1), jnp.float32)))

    tp = jnp.sum(matched, axis=1, keepdims=True)
    prec = tp / (float(N_GT) + fp)
    w = jnp.where(lax.broadcasted_iota(jnp.int32, (8, 1), 0) < N_THR,
                  1.0 / N_THR, 0.0)
    out_ref[0, 0] = jnp.sum(prec * w)


def _run_tc(pred_boxes, gt_boxes, interpret=False):
    gt_pad = jnp.concatenate(
        [gt_boxes, jnp.full((N_GT_PAD - N_GT, 4), 1e9, jnp.float32)], axis=0)
    args = [pred_boxes[:, k:k + 1] for k in range(4)]
    args += [gt_pad[:, k].reshape(1, N_GT_PAD) for k in range(4)]
    thr = np.full((8, 1), 2.0, np.float32)
    thr[:N_THR, 0] = _THR
    args.append(jnp.asarray(thr))
    out = pl.pallas_call(
        _tc_body,
        out_shape=jax.ShapeDtypeStruct((1, 1), jnp.float32),
        scratch_shapes=[pltpu.VMEM((N_PRED, N_GT_PAD), jnp.float32)],
        interpret=interpret,
    )(*args)
    return out[0, 0]


def kernel(pred_boxes, gt_boxes):
    return _run_tc(pred_boxes, gt_boxes)


# speedup vs baseline: 100.0949x; 100.0949x over previous
import numpy as np
import jax
import jax.numpy as jnp
from jax import lax
from jax.experimental import pallas as pl
from jax.experimental.pallas import tpu as pltpu

_THR = np.arange(0.5, 0.75, 0.05).astype(np.float32)
N_PRED = 5000
N_GT = 1000
N_GT_PAD = 1024
N_THR = len(_THR)


def _tc_body(px0, py0, px1, py1, gx0, gy0, gx1, gy1, thr, out_ref, mat_ref):
    g0x = gx0[...]
    g0y = gy0[...]
    g1x = gx1[...]
    g1y = gy1[...]
    garea = (g1x - g0x) * (g1y - g0y)

    def blk(b, _):
        r = b * 8
        p0x = px0[pl.ds(r, 8), :]
        p0y = py0[pl.ds(r, 8), :]
        p1x = px1[pl.ds(r, 8), :]
        p1y = py1[pl.ds(r, 8), :]
        parea = (p1x - p0x) * (p1y - p0y)
        ltx = jnp.maximum(p0x, g0x)
        lty = jnp.maximum(p0y, g0y)
        rbx = jnp.minimum(p1x, g1x)
        rby = jnp.minimum(p1y, g1y)
        whx = jnp.maximum(rbx - ltx, 0.0)
        why = jnp.maximum(rby - lty, 0.0)
        inter = whx * why
        union = parea + garea - inter
        mat_ref[pl.ds(r, 8), :] = inter / union
        return 0

    lax.fori_loop(0, N_PRED // 8, blk, 0)

    thrv = thr[...]
    col = lax.broadcasted_iota(jnp.int32, (8, N_GT_PAD), 1)

    def step(i, carry):
        matched, fp = carry
        row = mat_ref[pl.ds(i, 1), :]
        rowb = jnp.broadcast_to(row, (8, N_GT_PAD))
        masked = jnp.where(matched > 0.0, -1.0, rowb)
        mx = jnp.max(masked, axis=1, keepdims=True)
        ism = mx >= thrv
        idx = jnp.min(jnp.where(masked == mx, col, N_GT_PAD * 2), axis=1,
                      keepdims=True)
        matched = jnp.where(ism & (col == idx), 1.0, matched)
        fp = fp + jnp.where(ism, 0.0, 1.0)
        return matched, fp

    matched, fp = lax.fori_loop(
        0, N_PRED, step,
        (jnp.zeros((8, N_GT_PAD), jnp.float32), jnp.zeros((8, 1), jnp.float32)))

    tp = jnp.sum(matched, axis=1, keepdims=True)
    prec = tp / (float(N_GT) + fp)
    w = jnp.where(lax.broadcasted_iota(jnp.int32, (8, 1), 0) < N_THR,
                  1.0 / N_THR, 0.0)
    out_ref[...] = jnp.sum(prec * w, axis=(0, 1), keepdims=True)


def _run_tc(pred_boxes, gt_boxes, interpret=False):
    gt_pad = jnp.concatenate(
        [gt_boxes, jnp.full((N_GT_PAD - N_GT, 4), 1e9, jnp.float32)], axis=0)
    args = [pred_boxes[:, k:k + 1] for k in range(4)]
    args += [gt_pad[:, k].reshape(1, N_GT_PAD) for k in range(4)]
    thr = np.full((8, 1), 2.0, np.float32)
    thr[:N_THR, 0] = _THR
    args.append(jnp.asarray(thr))
    out = pl.pallas_call(
        _tc_body,
        out_shape=jax.ShapeDtypeStruct((1, 1), jnp.float32),
        scratch_shapes=[pltpu.VMEM((N_PRED, N_GT_PAD), jnp.float32)],
        interpret=interpret,
    )(*args)
    return out[0, 0]


def kernel(pred_boxes, gt_boxes):
    return _run_tc(pred_boxes, gt_boxes)


# speedup vs baseline: 233.0924x; 2.3287x over previous
import functools
import numpy as np
import jax
import jax.numpy as jnp
from jax import lax
from jax.experimental import pallas as pl
from jax.experimental.pallas import tpu as pltpu
from jax.experimental.pallas import tpu_sc as plsc

_THR = np.arange(0.5, 0.75, 0.05).astype(np.float32)
N_PRED = 5000
N_PP = 5008
N_GT = 1000
N_GT_PAD = 1024
N_THR = len(_THR)
_BIG = 1e9


def _tc_top2(px0, py0, px1, py1, gx0, gy0, gx1, gy1,
             v0o, v1o, i0o, i1o):
    g0x = gx0[...]
    g0y = gy0[...]
    g1x = gx1[...]
    g1y = gy1[...]
    garea = (g1x - g0x) * (g1y - g0y)
    col = lax.broadcasted_iota(jnp.int32, (8, N_GT_PAD), 1)

    def blk(b, _):
        r = b * 8
        p0x = px0[pl.ds(r, 8), :]
        p0y = py0[pl.ds(r, 8), :]
        p1x = px1[pl.ds(r, 8), :]
        p1y = py1[pl.ds(r, 8), :]
        parea = (p1x - p0x) * (p1y - p0y)
        ltx = jnp.maximum(p0x, g0x)
        lty = jnp.maximum(p0y, g0y)
        rbx = jnp.minimum(p1x, g1x)
        rby = jnp.minimum(p1y, g1y)
        whx = jnp.maximum(rbx - ltx, 0.0)
        why = jnp.maximum(rby - lty, 0.0)
        inter = whx * why
        iou = inter / (parea + garea - inter)
        mx0 = jnp.max(iou, axis=1, keepdims=True)
        id0 = jnp.min(jnp.where(iou == mx0, col, 2 * N_GT_PAD), axis=1,
                      keepdims=True)
        m2 = jnp.where(col == id0, -1.0, iou)
        mx1 = jnp.max(m2, axis=1, keepdims=True)
        id1 = jnp.min(jnp.where(m2 == mx1, col, 2 * N_GT_PAD), axis=1,
                      keepdims=True)
        v0o[pl.ds(r, 8), :] = mx0
        v1o[pl.ds(r, 8), :] = mx1
        i0o[pl.ds(r, 8), :] = id0
        i1o[pl.ds(r, 8), :] = id1
        return 0

    lax.fori_loop(0, N_PP // 8, blk, 0)


def _sc_greedy(v0h, v1h, i0h, i1h, px0h, py0h, px1h, py1h,
               gx0h, gy0h, gx1h, gy1h, thrh, outh,
               v0, v1, i0, i1, px0, py0, px1, py1,
               g0x, g0y, g1x, g1y, thrv, matched_v, matched_s,
               outv, accv, sem):
    c = lax.axis_index("c")
    s = lax.axis_index("s")
    is_worker = jnp.logical_and(c == 0, s < N_THR)
    lane = lax.broadcasted_iota(jnp.int32, (16,), 0)

    @pl.when(is_worker)
    def _work():
        copies = [
            pltpu.async_copy(v0h, v0, sem),
            pltpu.async_copy(v1h, v1, sem),
            pltpu.async_copy(i0h, i0, sem),
            pltpu.async_copy(i1h, i1, sem),
            pltpu.async_copy(px0h, px0, sem),
            pltpu.async_copy(py0h, py0, sem),
            pltpu.async_copy(px1h, px1, sem),
            pltpu.async_copy(py1h, py1, sem),
            pltpu.async_copy(gx0h, g0x, sem),
            pltpu.async_copy(gy0h, g0y, sem),
            pltpu.async_copy(gx1h, g1x, sem),
            pltpu.async_copy(gy1h, g1y, sem),
            pltpu.async_copy(thrh, thrv, sem),
        ]
        for cp in copies:
            cp.wait()

        thrc = thrv[pl.ds(s * 16, 16)]
        thr = thrc[0]

        def zero_v(k, _):
            matched_v[pl.ds(k * 16, 16)] = jnp.zeros((16,), jnp.float32)
            return 0

        lax.fori_loop(0, N_GT_PAD // 16, zero_v, 0)

        def zero_s(k, _):
            matched_s[k] = 0
            return 0

        lax.fori_loop(0, N_GT_PAD, zero_s, 0)

        def set_match(idx):
            matched_s[idx] = 1
            plsc.store_scatter(
                matched_v,
                [jnp.zeros((16,), jnp.int32) + idx],
                jnp.ones((16,), jnp.float32),
                mask=lane == 0)

        def rescan(p0x, p0y, p1x, p1y):
            parea = (p1x - p0x) * (p1y - p0y)

            def chunk(k, carry):
                bestv, besti = carry
                off = k * 16
                cg0x = g0x[pl.ds(off, 16)]
                cg0y = g0y[pl.ds(off, 16)]
                cg1x = g1x[pl.ds(off, 16)]
                cg1y = g1y[pl.ds(off, 16)]
                garea = (cg1x - cg0x) * (cg1y - cg0y)
                ltx = jnp.maximum(p0x, cg0x)
                lty = jnp.maximum(p0y, cg0y)
                rbx = jnp.minimum(p1x, cg1x)
                rby = jnp.minimum(p1y, cg1y)
                whx = jnp.maximum(rbx - ltx, 0.0)
                why = jnp.maximum(rby - lty, 0.0)
                inter = whx * why
                iou = inter / (parea + garea - inter)
                mch = matched_v[pl.ds(off, 16)]
                mskd = jnp.where(mch > 0.0, -1.0, iou)
                upd = mskd > bestv
                return (jnp.where(upd, mskd, bestv),
                        jnp.where(upd, lane + off, besti))

            bestv, besti = lax.fori_loop(
                0, N_GT_PAD // 16, chunk,
                (jnp.full((16,), -2.0, jnp.float32),
                 jnp.zeros((16,), jnp.int32)))
            sv, _si = plsc.sort_key_val(bestv, besti, descending=True)
            m = sv[0]

            @pl.when(m >= thr)
            def _():
                cand = jnp.where(bestv == m, besti, 2 * N_GT_PAD)
                ci, _cv = plsc.sort_key_val(cand, cand)
                set_match(ci[0])

        def chunkrow(k, _):
            base = k * 16
            v0c = v0[pl.ds(base, 16)]
            ncand = plsc.all_reduce_population_count(v0c >= thr)

            @pl.when(ncand[0] > 0)
            def _cands():
                v1c = v1[pl.ds(base, 16)]
                i0c = i0[pl.ds(base, 16)]
                i1c = i1[pl.ds(base, 16)]
                px0c = px0[pl.ds(base, 16)]
                py0c = py0[pl.ds(base, 16)]
                px1c = px1[pl.ds(base, 16)]
                py1c = py1[pl.ds(base, 16)]
                for j in range(16):
                    @pl.when(v0c[j] >= thr)
                    def _cand(j=j):
                        idx0 = i0c[j]
                        m0 = matched_s[idx0]

                        @pl.when(m0 == 0)
                        def _():
                            set_match(idx0)

                        @pl.when(m0 != 0)
                        def _():
                            @pl.when(v1c[j] >= thr)
                            def _():
                                idx1 = i1c[j]
                                m1 = matched_s[idx1]

                                @pl.when(m1 == 0)
                                def _():
                                    set_match(idx1)

                                @pl.when(m1 != 0)
                                def _():
                                    rescan(px0c[j], py0c[j],
                                           px1c[j], py1c[j])

            return 0

        lax.fori_loop(0, N_PP // 16, chunkrow, 0)

        def acc(k, a):
            mch = matched_v[pl.ds(k * 16, 16)] > 0.0
            return a + plsc.all_reduce_population_count(mch)[0]

        tpc = lax.fori_loop(0, N_GT_PAD // 16, acc, jnp.int32(0))
        tpv = jnp.zeros((16,), jnp.float32) + tpc.astype(jnp.float32)
        outv[...] = tpv / (6000.0 - tpv)
        pltpu.sync_copy(outv, outh.at[s])

    plsc.subcore_barrier()

    @pl.when(jnp.logical_and(c == 0, s == 0))
    def _final():
        total = outv[...]
        for t in range(1, N_THR):
            pltpu.async_copy(outh.at[t], accv, sem).wait()
            total = total + accv[...]
        outv[...] = total * (1.0 / N_THR)
        pltpu.sync_copy(outv, outh.at[0])


def kernel(pred_boxes, gt_boxes):
    pad_box = jnp.tile(
        jnp.asarray([[0.0, 0.0, 1.0, 1.0]], jnp.float32), (N_PP - N_PRED, 1))
    pred_pad = jnp.concatenate([pred_boxes, pad_box], axis=0)
    gt_pad = jnp.concatenate(
        [gt_boxes, jnp.full((N_GT_PAD - N_GT, 4), _BIG, jnp.float32)], axis=0)
    tc_args = [pred_pad[:, k:k + 1] for k in range(4)]
    tc_args += [gt_pad[:, k].reshape(1, N_GT_PAD) for k in range(4)]
    v0, v1, i0, i1 = pl.pallas_call(
        _tc_top2,
        out_shape=[
            jax.ShapeDtypeStruct((N_PP, 1), jnp.float32),
            jax.ShapeDtypeStruct((N_PP, 1), jnp.float32),
            jax.ShapeDtypeStruct((N_PP, 1), jnp.int32),
            jax.ShapeDtypeStruct((N_PP, 1), jnp.int32),
        ],
    )(*tc_args)

    thr = np.full((16, 16), 2.0, np.float32)
    thr[:N_THR, :] = _THR[:, None]
    mesh = plsc.VectorSubcoreMesh(core_axis_name="c", subcore_axis_name="s")
    sck = functools.partial(
        pl.kernel,
        mesh=mesh,
        out_type=jax.ShapeDtypeStruct((8, 16), jnp.float32),
        compiler_params=pltpu.CompilerParams(needs_layout_passes=False),
        scratch_types=[
            pltpu.VMEM((N_PP,), jnp.float32),
            pltpu.VMEM((N_PP,), jnp.float32),
            pltpu.VMEM((N_PP,), jnp.int32),
            pltpu.VMEM((N_PP,), jnp.int32),
            pltpu.VMEM((N_PP,), jnp.float32),
            pltpu.VMEM((N_PP,), jnp.float32),
            pltpu.VMEM((N_PP,), jnp.float32),
            pltpu.VMEM((N_PP,), jnp.float32),
            pltpu.VMEM((N_GT_PAD,), jnp.float32),
            pltpu.VMEM((N_GT_PAD,), jnp.float32),
            pltpu.VMEM((N_GT_PAD,), jnp.float32),
            pltpu.VMEM((N_GT_PAD,), jnp.float32),
            pltpu.VMEM((16 * 16,), jnp.float32),
            pltpu.VMEM((N_GT_PAD,), jnp.float32),
            pltpu.SMEM((N_GT_PAD,), jnp.int32),
            pltpu.VMEM((16,), jnp.float32),
            pltpu.VMEM((16,), jnp.float32),
            pltpu.SemaphoreType.DMA,
        ],
    )(_sc_greedy)
    out = sck(
        v0.reshape(N_PP), v1.reshape(N_PP),
        i0.reshape(N_PP), i1.reshape(N_PP),
        pred_pad[:, 0], pred_pad[:, 1], pred_pad[:, 2], pred_pad[:, 3],
        gt_pad[:, 0], gt_pad[:, 1], gt_pad[:, 2], gt_pad[:, 3],
        jnp.asarray(thr.reshape(16 * 16)),
    )
    return out[0, 0]


# speedup vs baseline: 438.7500x; 1.8823x over previous
import functools
import numpy as np
import jax
import jax.numpy as jnp
from jax import lax
from jax.experimental import pallas as pl
from jax.experimental.pallas import tpu as pltpu
from jax.experimental.pallas import tpu_sc as plsc

_THR = np.arange(0.5, 0.75, 0.05).astype(np.float32)
N_PRED = 5000
N_PP = 5008
N_GT = 1000
N_GT_PAD = 1024
N_THR = len(_THR)
_BIG = 1e9


_RB = 40


def _tc_top2(pred, gx0, gy0, gx1, gy1, v0o, v1o, i0o, i1o):
    g0x = gx0[...]
    g0y = gy0[...]
    g1x = gx1[...]
    g1y = gy1[...]
    garea = (g1x - g0x) * (g1y - g0y)
    col = lax.broadcasted_iota(jnp.int32, (_RB, N_GT_PAD), 1)

    def blk(b, _):
        r = b * _RB
        pb = pred[pl.ds(r, _RB), :]
        p0x = pb[:, 0:1]
        p0y = pb[:, 1:2]
        p1x = pb[:, 2:3]
        p1y = pb[:, 3:4]
        parea = (p1x - p0x) * (p1y - p0y)
        ltx = jnp.maximum(p0x, g0x)
        lty = jnp.maximum(p0y, g0y)
        rbx = jnp.minimum(p1x, g1x)
        rby = jnp.minimum(p1y, g1y)
        whx = jnp.maximum(rbx - ltx, 0.0)
        why = jnp.maximum(rby - lty, 0.0)
        inter = whx * why
        iou = inter / (parea + garea - inter)
        mx0 = jnp.max(iou, axis=1, keepdims=True)
        id0 = jnp.min(jnp.where(iou == mx0, col, 2 * N_GT_PAD), axis=1,
                      keepdims=True)
        m2 = jnp.where(col == id0, -1.0, iou)
        mx1 = jnp.max(m2, axis=1, keepdims=True)
        id1 = jnp.min(jnp.where(m2 == mx1, col, 2 * N_GT_PAD), axis=1,
                      keepdims=True)
        v0o[pl.ds(r, _RB), :] = mx0
        v1o[pl.ds(r, _RB), :] = mx1
        i0o[pl.ds(r, _RB), :] = id0
        i1o[pl.ds(r, _RB), :] = id1
        return 0

    lax.fori_loop(0, N_PRED // _RB, blk, 0)
    v0o[pl.ds(N_PRED, 8), :] = jnp.zeros((8, 1), jnp.float32)
    v1o[pl.ds(N_PRED, 8), :] = jnp.zeros((8, 1), jnp.float32)
    i0o[pl.ds(N_PRED, 8), :] = jnp.zeros((8, 1), jnp.int32)
    i1o[pl.ds(N_PRED, 8), :] = jnp.zeros((8, 1), jnp.int32)


def _sc_greedy(v0h, v1h, i0h, i1h, px0h, py0h, px1h, py1h,
               gx0h, gy0h, gx1h, gy1h, thrh, outh,
               v0, v1, i0, i1, px0, py0, px1, py1,
               g0x, g0y, g1x, g1y, thrv, matched_v, matched_s,
               outv, accv, sem):
    c = lax.axis_index("c")
    s = lax.axis_index("s")
    is_worker = jnp.logical_and(c == 0, s < N_THR)
    lane = lax.broadcasted_iota(jnp.int32, (16,), 0)

    @pl.when(is_worker)
    def _work():
        copies = [
            pltpu.async_copy(v0h, v0, sem),
            pltpu.async_copy(v1h, v1, sem),
            pltpu.async_copy(i0h, i0, sem),
            pltpu.async_copy(i1h, i1, sem),
            pltpu.async_copy(px0h, px0, sem),
            pltpu.async_copy(py0h, py0, sem),
            pltpu.async_copy(px1h, px1, sem),
            pltpu.async_copy(py1h, py1, sem),
            pltpu.async_copy(gx0h, g0x, sem),
            pltpu.async_copy(gy0h, g0y, sem),
            pltpu.async_copy(gx1h, g1x, sem),
            pltpu.async_copy(gy1h, g1y, sem),
            pltpu.async_copy(thrh, thrv, sem),
        ]
        for cp in copies:
            cp.wait()

        thrc = thrv[pl.ds(s * 16, 16)]
        thr = thrc[0]

        def zero_v(k, _):
            matched_v[pl.ds(k * 16, 16)] = jnp.zeros((16,), jnp.float32)
            return 0

        lax.fori_loop(0, N_GT_PAD // 16, zero_v, 0)

        def zero_s(k, _):
            matched_s[k] = 0
            return 0

        lax.fori_loop(0, N_GT_PAD, zero_s, 0)

        def set_match(idx):
            matched_s[idx] = 1
            plsc.store_scatter(
                matched_v,
                [jnp.zeros((16,), jnp.int32) + idx],
                jnp.ones((16,), jnp.float32),
                mask=lane == 0)

        def rescan(p0x, p0y, p1x, p1y):
            parea = (p1x - p0x) * (p1y - p0y)

            def chunk(k, carry):
                bestv, besti = carry
                off = k * 16
                cg0x = g0x[pl.ds(off, 16)]
                cg0y = g0y[pl.ds(off, 16)]
                cg1x = g1x[pl.ds(off, 16)]
                cg1y = g1y[pl.ds(off, 16)]
                garea = (cg1x - cg0x) * (cg1y - cg0y)
                ltx = jnp.maximum(p0x, cg0x)
                lty = jnp.maximum(p0y, cg0y)
                rbx = jnp.minimum(p1x, cg1x)
                rby = jnp.minimum(p1y, cg1y)
                whx = jnp.maximum(rbx - ltx, 0.0)
                why = jnp.maximum(rby - lty, 0.0)
                inter = whx * why
                iou = inter / (parea + garea - inter)
                mch = matched_v[pl.ds(off, 16)]
                mskd = jnp.where(mch > 0.0, -1.0, iou)
                upd = mskd > bestv
                return (jnp.where(upd, mskd, bestv),
                        jnp.where(upd, lane + off, besti))

            bestv, besti = lax.fori_loop(
                0, N_GT_PAD // 16, chunk,
                (jnp.full((16,), -2.0, jnp.float32),
                 jnp.zeros((16,), jnp.int32)))
            sv, _si = plsc.sort_key_val(bestv, besti, descending=True)
            m = sv[0]

            @pl.when(m >= thr)
            def _():
                cand = jnp.where(bestv == m, besti, 2 * N_GT_PAD)
                ci, _cv = plsc.sort_key_val(cand, cand)
                set_match(ci[0])

        def chunkrow(k, _):
            base = k * 16
            v0c = v0[pl.ds(base, 16)]
            ncand = plsc.all_reduce_population_count(v0c >= thr)

            @pl.when(ncand[0] > 0)
            def _cands():
                v1c = v1[pl.ds(base, 16)]
                i0c = i0[pl.ds(base, 16)]
                i1c = i1[pl.ds(base, 16)]
                px0c = px0[pl.ds(base, 16)]
                py0c = py0[pl.ds(base, 16)]
                px1c = px1[pl.ds(base, 16)]
                py1c = py1[pl.ds(base, 16)]
                for j in range(16):
                    @pl.when(v0c[j] >= thr)
                    def _cand(j=j):
                        idx0 = i0c[j]
                        m0 = matched_s[idx0]

                        @pl.when(m0 == 0)
                        def _():
                            set_match(idx0)

                        @pl.when(m0 != 0)
                        def _():
                            @pl.when(v1c[j] >= thr)
                            def _():
                                idx1 = i1c[j]
                                m1 = matched_s[idx1]

                                @pl.when(m1 == 0)
                                def _():
                                    set_match(idx1)

                                @pl.when(m1 != 0)
                                def _():
                                    rescan(px0c[j], py0c[j],
                                           px1c[j], py1c[j])

            return 0

        lax.fori_loop(0, N_PP // 16, chunkrow, 0)

        def acc(k, a):
            mch = matched_v[pl.ds(k * 16, 16)] > 0.0
            return a + plsc.all_reduce_population_count(mch)[0]

        tpc = lax.fori_loop(0, N_GT_PAD // 16, acc, jnp.int32(0))
        tpv = jnp.zeros((16,), jnp.float32) + tpc.astype(jnp.float32)
        outv[...] = tpv / (6000.0 - tpv)
        pltpu.sync_copy(outv, outh.at[s])

    plsc.subcore_barrier()

    @pl.when(jnp.logical_and(c == 0, s == 0))
    def _final():
        total = outv[...]
        for t in range(1, N_THR):
            pltpu.async_copy(outh.at[t], accv, sem).wait()
            total = total + accv[...]
        outv[...] = total * (1.0 / N_THR)
        pltpu.sync_copy(outv, outh.at[0])


def kernel(pred_boxes, gt_boxes):
    pad_box = jnp.tile(
        jnp.asarray([[0.0, 0.0, 1.0, 1.0]], jnp.float32), (N_PP - N_PRED, 1))
    pred_pad = jnp.concatenate([pred_boxes, pad_box], axis=0)
    gt_pad = jnp.concatenate(
        [gt_boxes, jnp.full((N_GT_PAD - N_GT, 4), _BIG, jnp.float32)], axis=0)
    tc_args = [pred_boxes]
    tc_args += [gt_pad[:, k].reshape(1, N_GT_PAD) for k in range(4)]
    v0, v1, i0, i1 = pl.pallas_call(
        _tc_top2,
        out_shape=[
            jax.ShapeDtypeStruct((N_PP, 1), jnp.float32),
            jax.ShapeDtypeStruct((N_PP, 1), jnp.float32),
            jax.ShapeDtypeStruct((N_PP, 1), jnp.int32),
            jax.ShapeDtypeStruct((N_PP, 1), jnp.int32),
        ],
    )(*tc_args)

    thr = np.full((16, 16), 2.0, np.float32)
    thr[:N_THR, :] = _THR[:, None]
    mesh = plsc.VectorSubcoreMesh(core_axis_name="c", subcore_axis_name="s")
    sck = functools.partial(
        pl.kernel,
        mesh=mesh,
        out_type=jax.ShapeDtypeStruct((8, 16), jnp.float32),
        compiler_params=pltpu.CompilerParams(needs_layout_passes=False),
        scratch_types=[
            pltpu.VMEM((N_PP,), jnp.float32),
            pltpu.VMEM((N_PP,), jnp.float32),
            pltpu.VMEM((N_PP,), jnp.int32),
            pltpu.VMEM((N_PP,), jnp.int32),
            pltpu.VMEM((N_PP,), jnp.float32),
            pltpu.VMEM((N_PP,), jnp.float32),
            pltpu.VMEM((N_PP,), jnp.float32),
            pltpu.VMEM((N_PP,), jnp.float32),
            pltpu.VMEM((N_GT_PAD,), jnp.float32),
            pltpu.VMEM((N_GT_PAD,), jnp.float32),
            pltpu.VMEM((N_GT_PAD,), jnp.float32),
            pltpu.VMEM((N_GT_PAD,), jnp.float32),
            pltpu.VMEM((16 * 16,), jnp.float32),
            pltpu.VMEM((N_GT_PAD,), jnp.float32),
            pltpu.SMEM((N_GT_PAD,), jnp.int32),
            pltpu.VMEM((16,), jnp.float32),
            pltpu.VMEM((16,), jnp.float32),
            pltpu.SemaphoreType.DMA,
        ],
    )(_sc_greedy)
    out = sck(
        v0.reshape(N_PP), v1.reshape(N_PP),
        i0.reshape(N_PP), i1.reshape(N_PP),
        pred_pad[:, 0], pred_pad[:, 1], pred_pad[:, 2], pred_pad[:, 3],
        gt_pad[:, 0], gt_pad[:, 1], gt_pad[:, 2], gt_pad[:, 3],
        jnp.asarray(thr.reshape(16 * 16)),
    )
    return out[0, 0]


# speedup vs baseline: 856.4594x; 1.9520x over previous
import functools
import numpy as np
import jax
import jax.numpy as jnp
from jax import lax
from jax.experimental import pallas as pl
from jax.experimental.pallas import tpu as pltpu
from jax.experimental.pallas import tpu_sc as plsc

_THR = np.arange(0.5, 0.75, 0.05).astype(np.float32)
N_PRED = 5000
N_PP = 5008
N_GT = 1000
N_GT_PAD = 1024
N_THR = len(_THR)
_BIG = 1e9


_RB = 40


def _tc_top2(pred, gx0, gy0, gx1, gy1, v0o, v1o, i0o, i1o):
    g0x = gx0[...]
    g0y = gy0[...]
    g1x = gx1[...]
    g1y = gy1[...]
    garea = (g1x - g0x) * (g1y - g0y)
    col = lax.broadcasted_iota(jnp.int32, (_RB, N_GT_PAD), 1)

    def blk(b, _):
        r = b * _RB
        pb = pred[pl.ds(r, _RB), :]
        p0x = pb[:, 0:1]
        p0y = pb[:, 1:2]
        p1x = pb[:, 2:3]
        p1y = pb[:, 3:4]
        parea = (p1x - p0x) * (p1y - p0y)
        ltx = jnp.maximum(p0x, g0x)
        lty = jnp.maximum(p0y, g0y)
        rbx = jnp.minimum(p1x, g1x)
        rby = jnp.minimum(p1y, g1y)
        whx = jnp.maximum(rbx - ltx, 0.0)
        why = jnp.maximum(rby - lty, 0.0)
        inter = whx * why
        iou = inter / (parea + garea - inter)
        mx0 = jnp.max(iou, axis=1, keepdims=True)
        id0 = jnp.min(jnp.where(iou == mx0, col, 2 * N_GT_PAD), axis=1,
                      keepdims=True)
        m2 = jnp.where(col == id0, -1.0, iou)
        mx1 = jnp.max(m2, axis=1, keepdims=True)
        id1 = jnp.min(jnp.where(m2 == mx1, col, 2 * N_GT_PAD), axis=1,
                      keepdims=True)
        v0o[pl.ds(r, _RB), :] = mx0
        v1o[pl.ds(r, _RB), :] = mx1
        i0o[pl.ds(r, _RB), :] = id0
        i1o[pl.ds(r, _RB), :] = id1
        return 0

    lax.fori_loop(0, N_PRED // _RB, blk, 0)
    v0o[pl.ds(N_PRED, 8), :] = jnp.zeros((8, 1), jnp.float32)
    v1o[pl.ds(N_PRED, 8), :] = jnp.zeros((8, 1), jnp.float32)
    i0o[pl.ds(N_PRED, 8), :] = jnp.zeros((8, 1), jnp.int32)
    i1o[pl.ds(N_PRED, 8), :] = jnp.zeros((8, 1), jnp.int32)


def _sc_greedy(v0h, v1h, i0h, i1h, px0h, py0h, px1h, py1h,
               gx0h, gy0h, gx1h, gy1h, thrh, outh,
               v0, v1, i0, i1, px0, py0, px1, py1,
               g0x, g0y, g1x, g1y, thrv, matched_v, matched_s,
               outv, accv, sem):
    c = lax.axis_index("c")
    s = lax.axis_index("s")
    is_worker = jnp.logical_and(c == 0, s < N_THR)
    lane = lax.broadcasted_iota(jnp.int32, (16,), 0)

    @pl.when(is_worker)
    def _work():
        copies = [
            pltpu.async_copy(v0h, v0, sem),
            pltpu.async_copy(v1h, v1, sem),
            pltpu.async_copy(i0h, i0, sem),
            pltpu.async_copy(i1h, i1, sem),
            pltpu.async_copy(px0h, px0, sem),
            pltpu.async_copy(py0h, py0, sem),
            pltpu.async_copy(px1h, px1, sem),
            pltpu.async_copy(py1h, py1, sem),
            pltpu.async_copy(gx0h, g0x, sem),
            pltpu.async_copy(gy0h, g0y, sem),
            pltpu.async_copy(gx1h, g1x, sem),
            pltpu.async_copy(gy1h, g1y, sem),
            pltpu.async_copy(thrh, thrv, sem),
        ]
        with jax.named_scope("sc_dma_wait"):
            for cp in copies:
                cp.wait()

        thrc = thrv[pl.ds(s * 16, 16)]
        thr = thrc[0]

        def zero_v(k, _):
            matched_v[pl.ds(k * 16, 16)] = jnp.zeros((16,), jnp.float32)
            return 0

        with jax.named_scope("sc_zero"):
            lax.fori_loop(0, N_GT_PAD // 16, zero_v, 0)

            def zero_s(k, _):
                matched_s[k] = 0
                return 0

            lax.fori_loop(0, N_GT_PAD, zero_s, 0)

        def set_match(idx):
            matched_s[idx] = 1
            plsc.store_scatter(
                matched_v,
                [jnp.zeros((16,), jnp.int32) + idx],
                jnp.ones((16,), jnp.float32),
                mask=lane == 0)

        def rescan(p0x, p0y, p1x, p1y):
            parea = (p1x - p0x) * (p1y - p0y)

            def chunk(k, carry):
                bestv, besti = carry
                off = k * 16
                cg0x = g0x[pl.ds(off, 16)]
                cg0y = g0y[pl.ds(off, 16)]
                cg1x = g1x[pl.ds(off, 16)]
                cg1y = g1y[pl.ds(off, 16)]
                garea = (cg1x - cg0x) * (cg1y - cg0y)
                ltx = jnp.maximum(p0x, cg0x)
                lty = jnp.maximum(p0y, cg0y)
                rbx = jnp.minimum(p1x, cg1x)
                rby = jnp.minimum(p1y, cg1y)
                whx = jnp.maximum(rbx - ltx, 0.0)
                why = jnp.maximum(rby - lty, 0.0)
                inter = whx * why
                iou = inter / (parea + garea - inter)
                mch = matched_v[pl.ds(off, 16)]
                mskd = jnp.where(mch > 0.0, -1.0, iou)
                upd = mskd > bestv
                return (jnp.where(upd, mskd, bestv),
                        jnp.where(upd, lane + off, besti))

            bestv, besti = lax.fori_loop(
                0, N_GT_PAD // 16, chunk,
                (jnp.full((16,), -2.0, jnp.float32),
                 jnp.zeros((16,), jnp.int32)))
            sv, _si = plsc.sort_key_val(bestv, besti, descending=True)
            m = sv[0]

            @pl.when(m >= thr)
            def _():
                cand = jnp.where(bestv == m, besti, 2 * N_GT_PAD)
                ci, _cv = plsc.sort_key_val(cand, cand)
                set_match(ci[0])

        def chunkrow(k, _):
            base = k * 16
            v0c = v0[pl.ds(base, 16)]
            cmask = v0c >= thr
            ncand = plsc.all_reduce_population_count(cmask)

            @pl.when(ncand[0] > 0)
            def _cands():
                def one_cand(carry):
                    mask, n = carry
                    jv = plsc.all_reduce_ffs(mask)
                    giv = jv + base
                    idx0 = plsc.load_gather(i0, [giv])[0]
                    m0 = matched_s[idx0]

                    @pl.when(m0 == 0)
                    def _():
                        set_match(idx0)

                    @pl.when(m0 != 0)
                    def _():
                        val1 = plsc.load_gather(v1, [giv])[0]

                        @pl.when(val1 >= thr)
                        def _():
                            idx1 = plsc.load_gather(i1, [giv])[0]
                            m1 = matched_s[idx1]

                            @pl.when(m1 == 0)
                            def _():
                                set_match(idx1)

                            @pl.when(m1 != 0)
                            def _():
                                rescan(plsc.load_gather(px0, [giv])[0],
                                       plsc.load_gather(py0, [giv])[0],
                                       plsc.load_gather(px1, [giv])[0],
                                       plsc.load_gather(py1, [giv])[0])

                    return (jnp.logical_and(mask, lane != jv), n - 1)

                lax.while_loop(lambda c: c[1] > 0, one_cand,
                               (cmask, ncand[0]))

            return 0

        with jax.named_scope("sc_main"):
            lax.fori_loop(0, N_PP // 16, chunkrow, 0)

        def acc(k, a):
            mch = matched_v[pl.ds(k * 16, 16)] > 0.0
            return a + plsc.all_reduce_population_count(mch)[0]

        tpc = lax.fori_loop(0, N_GT_PAD // 16, acc, jnp.int32(0))
        tpv = jnp.zeros((16,), jnp.float32) + tpc.astype(jnp.float32)
        outv[...] = tpv / (6000.0 - tpv)
        pltpu.sync_copy(outv, outh.at[s])

    plsc.subcore_barrier()

    @pl.when(jnp.logical_and(c == 0, s == 0))
    def _final():
        total = outv[...]
        for t in range(1, N_THR):
            pltpu.async_copy(outh.at[t], accv, sem).wait()
            total = total + accv[...]
        outv[...] = total * (1.0 / N_THR)
        pltpu.sync_copy(outv, outh.at[0])


def kernel(pred_boxes, gt_boxes):
    pad_box = jnp.tile(
        jnp.asarray([[0.0, 0.0, 1.0, 1.0]], jnp.float32), (N_PP - N_PRED, 1))
    pred_pad = jnp.concatenate([pred_boxes, pad_box], axis=0)
    gt_pad = jnp.concatenate(
        [gt_boxes, jnp.full((N_GT_PAD - N_GT, 4), _BIG, jnp.float32)], axis=0)
    tc_args = [pred_boxes]
    tc_args += [gt_pad[:, k].reshape(1, N_GT_PAD) for k in range(4)]
    v0, v1, i0, i1 = pl.pallas_call(
        _tc_top2,
        out_shape=[
            jax.ShapeDtypeStruct((N_PP, 1), jnp.float32),
            jax.ShapeDtypeStruct((N_PP, 1), jnp.float32),
            jax.ShapeDtypeStruct((N_PP, 1), jnp.int32),
            jax.ShapeDtypeStruct((N_PP, 1), jnp.int32),
        ],
    )(*tc_args)

    thr = np.full((16, 16), 2.0, np.float32)
    thr[:N_THR, :] = _THR[:, None]
    mesh = plsc.VectorSubcoreMesh(core_axis_name="c", subcore_axis_name="s")
    sck = functools.partial(
        pl.kernel,
        mesh=mesh,
        out_type=jax.ShapeDtypeStruct((8, 16), jnp.float32),
        compiler_params=pltpu.CompilerParams(needs_layout_passes=False),
        scratch_types=[
            pltpu.VMEM((N_PP,), jnp.float32),
            pltpu.VMEM((N_PP,), jnp.float32),
            pltpu.VMEM((N_PP,), jnp.int32),
            pltpu.VMEM((N_PP,), jnp.int32),
            pltpu.VMEM((N_PP,), jnp.float32),
            pltpu.VMEM((N_PP,), jnp.float32),
            pltpu.VMEM((N_PP,), jnp.float32),
            pltpu.VMEM((N_PP,), jnp.float32),
            pltpu.VMEM((N_GT_PAD,), jnp.float32),
            pltpu.VMEM((N_GT_PAD,), jnp.float32),
            pltpu.VMEM((N_GT_PAD,), jnp.float32),
            pltpu.VMEM((N_GT_PAD,), jnp.float32),
            pltpu.VMEM((16 * 16,), jnp.float32),
            pltpu.VMEM((N_GT_PAD,), jnp.float32),
            pltpu.SMEM((N_GT_PAD,), jnp.int32),
            pltpu.VMEM((16,), jnp.float32),
            pltpu.VMEM((16,), jnp.float32),
            pltpu.SemaphoreType.DMA,
        ],
    )(_sc_greedy)
    out = sck(
        v0.reshape(N_PP), v1.reshape(N_PP),
        i0.reshape(N_PP), i1.reshape(N_PP),
        pred_pad[:, 0], pred_pad[:, 1], pred_pad[:, 2], pred_pad[:, 3],
        gt_pad[:, 0], gt_pad[:, 1], gt_pad[:, 2], gt_pad[:, 3],
        jnp.asarray(thr.reshape(16 * 16)),
    )
    return out[0, 0]


# speedup vs baseline: 1410.6913x; 1.6471x over previous
import functools
import numpy as np
import jax
import jax.numpy as jnp
from jax import lax
from jax.experimental import pallas as pl
from jax.experimental.pallas import tpu as pltpu
from jax.experimental.pallas import tpu_sc as plsc

_THR = np.arange(0.5, 0.75, 0.05).astype(np.float32)
N_PRED = 5000
N_PP = 5008
N_GT = 1000
N_GT_PAD = 1024
N_THR = len(_THR)
_BIG = 1e9


_RB = 200


def _tc_top2(pred, gx0, gy0, gx1, gy1, v0o, v1o, i0o, i1o):
    g0x = gx0[...]
    g0y = gy0[...]
    g1x = gx1[...]
    g1y = gy1[...]
    garea = (g1x - g0x) * (g1y - g0y)
    col = lax.broadcasted_iota(jnp.int32, (_RB, N_GT_PAD), 1)

    def blk(b, _):
        r = b * _RB
        pb = pred[pl.ds(r, _RB), :]
        p0x = pb[:, 0:1]
        p0y = pb[:, 1:2]
        p1x = pb[:, 2:3]
        p1y = pb[:, 3:4]
        parea = (p1x - p0x) * (p1y - p0y)
        ltx = jnp.maximum(p0x, g0x)
        lty = jnp.maximum(p0y, g0y)
        rbx = jnp.minimum(p1x, g1x)
        rby = jnp.minimum(p1y, g1y)
        whx = jnp.maximum(rbx - ltx, 0.0)
        why = jnp.maximum(rby - lty, 0.0)
        inter = whx * why
        iou = inter / (parea + garea - inter)
        mx0 = jnp.max(iou, axis=1, keepdims=True)
        id0 = jnp.min(jnp.where(iou == mx0, col, 2 * N_GT_PAD), axis=1,
                      keepdims=True)
        m2 = jnp.where(col == id0, -1.0, iou)
        mx1 = jnp.max(m2, axis=1, keepdims=True)
        id1 = jnp.min(jnp.where(m2 == mx1, col, 2 * N_GT_PAD), axis=1,
                      keepdims=True)
        v0o[pl.ds(r, _RB), :] = mx0
        v1o[pl.ds(r, _RB), :] = mx1
        i0o[pl.ds(r, _RB), :] = id0
        i1o[pl.ds(r, _RB), :] = id1
        return 0

    lax.fori_loop(0, N_PRED // _RB, blk, 0)
    v0o[pl.ds(N_PRED, 8), :] = jnp.zeros((8, 1), jnp.float32)
    v1o[pl.ds(N_PRED, 8), :] = jnp.zeros((8, 1), jnp.float32)
    i0o[pl.ds(N_PRED, 8), :] = jnp.zeros((8, 1), jnp.int32)
    i1o[pl.ds(N_PRED, 8), :] = jnp.zeros((8, 1), jnp.int32)


def _sc_greedy(v0h, v1h, i0h, i1h, px0h, py0h, px1h, py1h,
               gx0h, gy0h, gx1h, gy1h, thrh, outh,
               v0, v1, i0, i1, px0, py0, px1, py1,
               g0x, g0y, g1x, g1y, thrv, matched_v, matched_s,
               outv, accv, sem):
    c = lax.axis_index("c")
    s = lax.axis_index("s")
    is_worker = jnp.logical_and(c == 0, s < N_THR)
    lane = lax.broadcasted_iota(jnp.int32, (16,), 0)

    @pl.when(is_worker)
    def _work():
        copies = [
            pltpu.async_copy(v0h, v0, sem),
            pltpu.async_copy(v1h, v1, sem),
            pltpu.async_copy(i0h, i0, sem),
            pltpu.async_copy(i1h, i1, sem),
            pltpu.async_copy(px0h, px0, sem),
            pltpu.async_copy(py0h, py0, sem),
            pltpu.async_copy(px1h, px1, sem),
            pltpu.async_copy(py1h, py1, sem),
            pltpu.async_copy(gx0h, g0x, sem),
            pltpu.async_copy(gy0h, g0y, sem),
            pltpu.async_copy(gx1h, g1x, sem),
            pltpu.async_copy(gy1h, g1y, sem),
            pltpu.async_copy(thrh, thrv, sem),
        ]
        with jax.named_scope("sc_dma_wait"):
            for cp in copies:
                cp.wait()

        thrc = thrv[pl.ds(s * 16, 16)]
        thr = thrc[0]

        def zero_v(k, _):
            matched_v[pl.ds(k * 16, 16)] = jnp.zeros((16,), jnp.float32)
            return 0

        with jax.named_scope("sc_zero"):
            lax.fori_loop(0, N_GT_PAD // 16, zero_v, 0)

            def zero_s(k, _):
                matched_s[k] = 0
                return 0

            lax.fori_loop(0, N_GT_PAD, zero_s, 0)

        def set_match(idx):
            matched_s[idx] = 1
            plsc.store_scatter(
                matched_v,
                [jnp.zeros((16,), jnp.int32) + idx],
                jnp.ones((16,), jnp.float32),
                mask=lane == 0)

        def rescan(p0x, p0y, p1x, p1y):
            parea = (p1x - p0x) * (p1y - p0y)

            def chunk(k, carry):
                bestv, besti = carry
                off = k * 16
                cg0x = g0x[pl.ds(off, 16)]
                cg0y = g0y[pl.ds(off, 16)]
                cg1x = g1x[pl.ds(off, 16)]
                cg1y = g1y[pl.ds(off, 16)]
                garea = (cg1x - cg0x) * (cg1y - cg0y)
                ltx = jnp.maximum(p0x, cg0x)
                lty = jnp.maximum(p0y, cg0y)
                rbx = jnp.minimum(p1x, cg1x)
                rby = jnp.minimum(p1y, cg1y)
                whx = jnp.maximum(rbx - ltx, 0.0)
                why = jnp.maximum(rby - lty, 0.0)
                inter = whx * why
                iou = inter / (parea + garea - inter)
                mch = matched_v[pl.ds(off, 16)]
                mskd = jnp.where(mch > 0.0, -1.0, iou)
                upd = mskd > bestv
                return (jnp.where(upd, mskd, bestv),
                        jnp.where(upd, lane + off, besti))

            bestv, besti = lax.fori_loop(
                0, N_GT_PAD // 16, chunk,
                (jnp.full((16,), -2.0, jnp.float32),
                 jnp.zeros((16,), jnp.int32)))
            sv, _si = plsc.sort_key_val(bestv, besti, descending=True)
            m = sv[0]

            @pl.when(m >= thr)
            def _():
                cand = jnp.where(bestv == m, besti, 2 * N_GT_PAD)
                ci, _cv = plsc.sort_key_val(cand, cand)
                set_match(ci[0])

        def chunkrow(k, _):
            base = k * 16
            v0c = v0[pl.ds(base, 16)]
            cmask = v0c >= thr
            ncand = plsc.all_reduce_population_count(cmask)

            @pl.when(ncand[0] > 0)
            def _cands():
                def one_cand(carry):
                    mask, n = carry
                    jv = plsc.all_reduce_ffs(mask)
                    giv = jv + base
                    idx0 = plsc.load_gather(i0, [giv])[0]
                    m0 = matched_s[idx0]

                    @pl.when(m0 == 0)
                    def _():
                        set_match(idx0)

                    @pl.when(m0 != 0)
                    def _():
                        val1 = plsc.load_gather(v1, [giv])[0]

                        @pl.when(val1 >= thr)
                        def _():
                            idx1 = plsc.load_gather(i1, [giv])[0]
                            m1 = matched_s[idx1]

                            @pl.when(m1 == 0)
                            def _():
                                set_match(idx1)

                            @pl.when(m1 != 0)
                            def _():
                                rescan(plsc.load_gather(px0, [giv])[0],
                                       plsc.load_gather(py0, [giv])[0],
                                       plsc.load_gather(px1, [giv])[0],
                                       plsc.load_gather(py1, [giv])[0])

                    return (jnp.logical_and(mask, lane != jv), n - 1)

                lax.while_loop(lambda c: c[1] > 0, one_cand,
                               (cmask, ncand[0]))

            return 0

        with jax.named_scope("sc_main"):
            lax.fori_loop(0, N_PP // 16, chunkrow, 0)

        def acc(k, a):
            mch = matched_v[pl.ds(k * 16, 16)] > 0.0
            return a + plsc.all_reduce_population_count(mch)[0]

        tpc = lax.fori_loop(0, N_GT_PAD // 16, acc, jnp.int32(0))
        tpv = jnp.zeros((16,), jnp.float32) + tpc.astype(jnp.float32)
        outv[...] = tpv / (6000.0 - tpv)
        pltpu.sync_copy(outv, outh.at[s])

    plsc.subcore_barrier()

    @pl.when(jnp.logical_and(c == 0, s == 0))
    def _final():
        total = outv[...]
        for t in range(1, N_THR):
            pltpu.async_copy(outh.at[t], accv, sem).wait()
            total = total + accv[...]
        outv[...] = total * (1.0 / N_THR)
        pltpu.sync_copy(outv, outh.at[0])


def kernel(pred_boxes, gt_boxes):
    pad_box = jnp.tile(
        jnp.asarray([[0.0, 0.0, 1.0, 1.0]], jnp.float32), (N_PP - N_PRED, 1))
    pred_pad = jnp.concatenate([pred_boxes, pad_box], axis=0)
    gt_pad = jnp.concatenate(
        [gt_boxes, jnp.full((N_GT_PAD - N_GT, 4), _BIG, jnp.float32)], axis=0)
    tc_args = [pred_boxes]
    tc_args += [gt_pad[:, k].reshape(1, N_GT_PAD) for k in range(4)]
    v0, v1, i0, i1 = pl.pallas_call(
        _tc_top2,
        out_shape=[
            jax.ShapeDtypeStruct((N_PP, 1), jnp.float32),
            jax.ShapeDtypeStruct((N_PP, 1), jnp.float32),
            jax.ShapeDtypeStruct((N_PP, 1), jnp.int32),
            jax.ShapeDtypeStruct((N_PP, 1), jnp.int32),
        ],
    )(*tc_args)

    thr = np.full((16, 16), 2.0, np.float32)
    thr[:N_THR, :] = _THR[:, None]
    mesh = plsc.VectorSubcoreMesh(core_axis_name="c", subcore_axis_name="s")
    sck = functools.partial(
        pl.kernel,
        mesh=mesh,
        out_type=jax.ShapeDtypeStruct((8, 16), jnp.float32),
        compiler_params=pltpu.CompilerParams(needs_layout_passes=False),
        scratch_types=[
            pltpu.VMEM((N_PP,), jnp.float32),
            pltpu.VMEM((N_PP,), jnp.float32),
            pltpu.VMEM((N_PP,), jnp.int32),
            pltpu.VMEM((N_PP,), jnp.int32),
            pltpu.VMEM((N_PP,), jnp.float32),
            pltpu.VMEM((N_PP,), jnp.float32),
            pltpu.VMEM((N_PP,), jnp.float32),
            pltpu.VMEM((N_PP,), jnp.float32),
            pltpu.VMEM((N_GT_PAD,), jnp.float32),
            pltpu.VMEM((N_GT_PAD,), jnp.float32),
            pltpu.VMEM((N_GT_PAD,), jnp.float32),
            pltpu.VMEM((N_GT_PAD,), jnp.float32),
            pltpu.VMEM((16 * 16,), jnp.float32),
            pltpu.VMEM((N_GT_PAD,), jnp.float32),
            pltpu.SMEM((N_GT_PAD,), jnp.int32),
            pltpu.VMEM((16,), jnp.float32),
            pltpu.VMEM((16,), jnp.float32),
            pltpu.SemaphoreType.DMA,
        ],
    )(_sc_greedy)
    out = sck(
        v0.reshape(N_PP), v1.reshape(N_PP),
        i0.reshape(N_PP), i1.reshape(N_PP),
        pred_pad[:, 0], pred_pad[:, 1], pred_pad[:, 2], pred_pad[:, 3],
        gt_pad[:, 0], gt_pad[:, 1], gt_pad[:, 2], gt_pad[:, 3],
        jnp.asarray(thr.reshape(16 * 16)),
    )
    return out[0, 0]


# speedup vs baseline: 1497.7239x; 1.0617x over previous
import functools
import numpy as np
import jax
import jax.numpy as jnp
from jax import lax
from jax.experimental import pallas as pl
from jax.experimental.pallas import tpu as pltpu
from jax.experimental.pallas import tpu_sc as plsc

_THR = np.arange(0.5, 0.75, 0.05).astype(np.float32)
N_PRED = 5000
N_PP = 5008
N_GT = 1000
N_GT_PAD = 1024
N_THR = len(_THR)
_BIG = 1e9


_RB = 1000


def _tc_top2(pred, gx0, gy0, gx1, gy1, v0o, v1o, i0o, i1o):
    g0x = gx0[...]
    g0y = gy0[...]
    g1x = gx1[...]
    g1y = gy1[...]
    garea = (g1x - g0x) * (g1y - g0y)
    col = lax.broadcasted_iota(jnp.int32, (_RB, N_GT_PAD), 1)

    def blk(b, _):
        r = b * _RB
        pb = pred[pl.ds(r, _RB), :]
        p0x = pb[:, 0:1]
        p0y = pb[:, 1:2]
        p1x = pb[:, 2:3]
        p1y = pb[:, 3:4]
        parea = (p1x - p0x) * (p1y - p0y)
        ltx = jnp.maximum(p0x, g0x)
        lty = jnp.maximum(p0y, g0y)
        rbx = jnp.minimum(p1x, g1x)
        rby = jnp.minimum(p1y, g1y)
        whx = jnp.maximum(rbx - ltx, 0.0)
        why = jnp.maximum(rby - lty, 0.0)
        inter = whx * why
        iou = inter / (parea + garea - inter)
        mx0 = jnp.max(iou, axis=1, keepdims=True)
        id0 = jnp.min(jnp.where(iou == mx0, col, 2 * N_GT_PAD), axis=1,
                      keepdims=True)
        m2 = jnp.where(col == id0, -1.0, iou)
        mx1 = jnp.max(m2, axis=1, keepdims=True)
        id1 = jnp.min(jnp.where(m2 == mx1, col, 2 * N_GT_PAD), axis=1,
                      keepdims=True)
        v0o[pl.ds(r, _RB), :] = mx0
        v1o[pl.ds(r, _RB), :] = mx1
        i0o[pl.ds(r, _RB), :] = id0
        i1o[pl.ds(r, _RB), :] = id1
        return 0

    lax.fori_loop(0, N_PRED // _RB, blk, 0)
    v0o[pl.ds(N_PRED, 8), :] = jnp.zeros((8, 1), jnp.float32)
    v1o[pl.ds(N_PRED, 8), :] = jnp.zeros((8, 1), jnp.float32)
    i0o[pl.ds(N_PRED, 8), :] = jnp.zeros((8, 1), jnp.int32)
    i1o[pl.ds(N_PRED, 8), :] = jnp.zeros((8, 1), jnp.int32)


def _sc_greedy(v0h, v1h, i0h, i1h, px0h, py0h, px1h, py1h,
               gx0h, gy0h, gx1h, gy1h, thrh, outh,
               v0, v1, i0, i1, px0, py0, px1, py1,
               g0x, g0y, g1x, g1y, thrv, matched_v, matched_s,
               outv, accv, sem):
    c = lax.axis_index("c")
    s = lax.axis_index("s")
    is_worker = jnp.logical_and(c == 0, s < N_THR)
    lane = lax.broadcasted_iota(jnp.int32, (16,), 0)

    @pl.when(is_worker)
    def _work():
        copies = [
            pltpu.async_copy(v0h, v0, sem),
            pltpu.async_copy(v1h, v1, sem),
            pltpu.async_copy(i0h, i0, sem),
            pltpu.async_copy(i1h, i1, sem),
            pltpu.async_copy(px0h, px0, sem),
            pltpu.async_copy(py0h, py0, sem),
            pltpu.async_copy(px1h, px1, sem),
            pltpu.async_copy(py1h, py1, sem),
            pltpu.async_copy(gx0h, g0x, sem),
            pltpu.async_copy(gy0h, g0y, sem),
            pltpu.async_copy(gx1h, g1x, sem),
            pltpu.async_copy(gy1h, g1y, sem),
            pltpu.async_copy(thrh, thrv, sem),
        ]
        with jax.named_scope("sc_dma_wait"):
            for cp in copies:
                cp.wait()

        thrc = thrv[pl.ds(s * 16, 16)]
        thr = thrc[0]

        def zero_v(k, _):
            matched_v[pl.ds(k * 16, 16)] = jnp.zeros((16,), jnp.float32)
            return 0

        with jax.named_scope("sc_zero"):
            lax.fori_loop(0, N_GT_PAD // 16, zero_v, 0)

            def zero_s(k, _):
                matched_s[k] = 0
                return 0

            lax.fori_loop(0, N_GT_PAD, zero_s, 0)

        def set_match(idx):
            matched_s[idx] = 1
            plsc.store_scatter(
                matched_v,
                [jnp.zeros((16,), jnp.int32) + idx],
                jnp.ones((16,), jnp.float32),
                mask=lane == 0)

        def rescan(p0x, p0y, p1x, p1y):
            parea = (p1x - p0x) * (p1y - p0y)

            def chunk(k, carry):
                bestv, besti = carry
                off = k * 16
                cg0x = g0x[pl.ds(off, 16)]
                cg0y = g0y[pl.ds(off, 16)]
                cg1x = g1x[pl.ds(off, 16)]
                cg1y = g1y[pl.ds(off, 16)]
                garea = (cg1x - cg0x) * (cg1y - cg0y)
                ltx = jnp.maximum(p0x, cg0x)
                lty = jnp.maximum(p0y, cg0y)
                rbx = jnp.minimum(p1x, cg1x)
                rby = jnp.minimum(p1y, cg1y)
                whx = jnp.maximum(rbx - ltx, 0.0)
                why = jnp.maximum(rby - lty, 0.0)
                inter = whx * why
                iou = inter / (parea + garea - inter)
                mch = matched_v[pl.ds(off, 16)]
                mskd = jnp.where(mch > 0.0, -1.0, iou)
                upd = mskd > bestv
                return (jnp.where(upd, mskd, bestv),
                        jnp.where(upd, lane + off, besti))

            bestv, besti = lax.fori_loop(
                0, N_GT_PAD // 16, chunk,
                (jnp.full((16,), -2.0, jnp.float32),
                 jnp.zeros((16,), jnp.int32)))
            sv, _si = plsc.sort_key_val(bestv, besti, descending=True)
            m = sv[0]

            @pl.when(m >= thr)
            def _():
                cand = jnp.where(bestv == m, besti, 2 * N_GT_PAD)
                ci, _cv = plsc.sort_key_val(cand, cand)
                set_match(ci[0])

        def chunkrow(k, _):
            base = k * 16
            v0c = v0[pl.ds(base, 16)]
            cmask = v0c >= thr
            ncand = plsc.all_reduce_population_count(cmask)

            @pl.when(ncand[0] > 0)
            def _cands():
                def one_cand(carry):
                    mask, n = carry
                    jv = plsc.all_reduce_ffs(mask)
                    giv = jv + base
                    idx0 = plsc.load_gather(i0, [giv])[0]
                    m0 = matched_s[idx0]

                    @pl.when(m0 == 0)
                    def _():
                        set_match(idx0)

                    @pl.when(m0 != 0)
                    def _():
                        val1 = plsc.load_gather(v1, [giv])[0]

                        @pl.when(val1 >= thr)
                        def _():
                            idx1 = plsc.load_gather(i1, [giv])[0]
                            m1 = matched_s[idx1]

                            @pl.when(m1 == 0)
                            def _():
                                set_match(idx1)

                            @pl.when(m1 != 0)
                            def _():
                                rescan(plsc.load_gather(px0, [giv])[0],
                                       plsc.load_gather(py0, [giv])[0],
                                       plsc.load_gather(px1, [giv])[0],
                                       plsc.load_gather(py1, [giv])[0])

                    return (jnp.logical_and(mask, lane != jv), n - 1)

                lax.while_loop(lambda c: c[1] > 0, one_cand,
                               (cmask, ncand[0]))

            return 0

        with jax.named_scope("sc_main"):
            lax.fori_loop(0, N_PP // 16, chunkrow, 0)

        def acc(k, a):
            mch = matched_v[pl.ds(k * 16, 16)] > 0.0
            return a + plsc.all_reduce_population_count(mch)[0]

        tpc = lax.fori_loop(0, N_GT_PAD // 16, acc, jnp.int32(0))
        tpv = jnp.zeros((16,), jnp.float32) + tpc.astype(jnp.float32)
        outv[...] = tpv / (6000.0 - tpv)
        pltpu.sync_copy(outv, outh.at[s])

    plsc.subcore_barrier()

    @pl.when(jnp.logical_and(c == 0, s == 0))
    def _final():
        total = outv[...]
        for t in range(1, N_THR):
            pltpu.async_copy(outh.at[t], accv, sem).wait()
            total = total + accv[...]
        outv[...] = total * (1.0 / N_THR)
        pltpu.sync_copy(outv, outh.at[0])


def kernel(pred_boxes, gt_boxes):
    pad_box = jnp.tile(
        jnp.asarray([[0.0, 0.0, 1.0, 1.0]], jnp.float32), (N_PP - N_PRED, 1))
    pred_pad = jnp.concatenate([pred_boxes, pad_box], axis=0)
    gt_pad = jnp.concatenate(
        [gt_boxes, jnp.full((N_GT_PAD - N_GT, 4), _BIG, jnp.float32)], axis=0)
    tc_args = [pred_boxes]
    tc_args += [gt_pad[:, k].reshape(1, N_GT_PAD) for k in range(4)]
    v0, v1, i0, i1 = pl.pallas_call(
        _tc_top2,
        out_shape=[
            jax.ShapeDtypeStruct((N_PP, 1), jnp.float32),
            jax.ShapeDtypeStruct((N_PP, 1), jnp.float32),
            jax.ShapeDtypeStruct((N_PP, 1), jnp.int32),
            jax.ShapeDtypeStruct((N_PP, 1), jnp.int32),
        ],
    )(*tc_args)

    thr = np.full((16, 16), 2.0, np.float32)
    thr[:N_THR, :] = _THR[:, None]
    mesh = plsc.VectorSubcoreMesh(core_axis_name="c", subcore_axis_name="s")
    sck = functools.partial(
        pl.kernel,
        mesh=mesh,
        out_type=jax.ShapeDtypeStruct((8, 16), jnp.float32),
        compiler_params=pltpu.CompilerParams(needs_layout_passes=False),
        scratch_types=[
            pltpu.VMEM((N_PP,), jnp.float32),
            pltpu.VMEM((N_PP,), jnp.float32),
            pltpu.VMEM((N_PP,), jnp.int32),
            pltpu.VMEM((N_PP,), jnp.int32),
            pltpu.VMEM((N_PP,), jnp.float32),
            pltpu.VMEM((N_PP,), jnp.float32),
            pltpu.VMEM((N_PP,), jnp.float32),
            pltpu.VMEM((N_PP,), jnp.float32),
            pltpu.VMEM((N_GT_PAD,), jnp.float32),
            pltpu.VMEM((N_GT_PAD,), jnp.float32),
            pltpu.VMEM((N_GT_PAD,), jnp.float32),
            pltpu.VMEM((N_GT_PAD,), jnp.float32),
            pltpu.VMEM((16 * 16,), jnp.float32),
            pltpu.VMEM((N_GT_PAD,), jnp.float32),
            pltpu.SMEM((N_GT_PAD,), jnp.int32),
            pltpu.VMEM((16,), jnp.float32),
            pltpu.VMEM((16,), jnp.float32),
            pltpu.SemaphoreType.DMA,
        ],
    )(_sc_greedy)
    out = sck(
        v0.reshape(N_PP), v1.reshape(N_PP),
        i0.reshape(N_PP), i1.reshape(N_PP),
        pred_pad[:, 0], pred_pad[:, 1], pred_pad[:, 2], pred_pad[:, 3],
        gt_pad[:, 0], gt_pad[:, 1], gt_pad[:, 2], gt_pad[:, 3],
        jnp.asarray(thr.reshape(16 * 16)),
    )
    return out[0, 0]


# speedup vs baseline: 1613.7937x; 1.0775x over previous
import functools
import numpy as np
import jax
import jax.numpy as jnp
from jax import lax
from jax.experimental import pallas as pl
from jax.experimental.pallas import tpu as pltpu
from jax.experimental.pallas import tpu_sc as plsc

_THR = np.arange(0.5, 0.75, 0.05).astype(np.float32)
N_PRED = 5000
N_PP = 5008
N_GT = 1000
N_GT_PAD = 1024
N_THR = len(_THR)
_BIG = 1e9
_RB = 1000


def _tc_top2(pred, gx0, gy0, gx1, gy1, v4o):
    g0x = gx0[...]
    g0y = gy0[...]
    g1x = gx1[...]
    g1y = gy1[...]
    garea = (g1x - g0x) * (g1y - g0y)
    col = lax.broadcasted_iota(jnp.int32, (_RB, N_GT_PAD), 1)

    def blk(b, _):
        r = b * _RB
        pb = pred[pl.ds(r, _RB), :]
        p0x = pb[:, 0:1]
        p0y = pb[:, 1:2]
        p1x = pb[:, 2:3]
        p1y = pb[:, 3:4]
        parea = (p1x - p0x) * (p1y - p0y)
        ltx = jnp.maximum(p0x, g0x)
        lty = jnp.maximum(p0y, g0y)
        rbx = jnp.minimum(p1x, g1x)
        rby = jnp.minimum(p1y, g1y)
        whx = jnp.maximum(rbx - ltx, 0.0)
        why = jnp.maximum(rby - lty, 0.0)
        inter = whx * why
        iou = inter / (parea + garea - inter)
        mx0 = jnp.max(iou, axis=1, keepdims=True)
        id0 = jnp.min(jnp.where(iou == mx0, col, 2 * N_GT_PAD), axis=1,
                      keepdims=True)
        m2 = jnp.where(col == id0, -1.0, iou)
        mx1 = jnp.max(m2, axis=1, keepdims=True)
        id1 = jnp.min(jnp.where(m2 == mx1, col, 2 * N_GT_PAD), axis=1,
                      keepdims=True)
        v4o[pl.ds(r, _RB), :] = jnp.concatenate(
            [mx0, mx1,
             lax.bitcast_convert_type(id0, jnp.float32),
             lax.bitcast_convert_type(id1, jnp.float32)], axis=1)
        return 0

    lax.fori_loop(0, N_PRED // _RB, blk, 0)
    v4o[pl.ds(N_PRED, 8), :] = jnp.zeros((8, 4), jnp.float32)


def _sc_greedy(v4h, predh, gth, thrh, outh,
               v4, predv, gtv, thrv, matched_v, matched_s,
               outv, accv, sem):
    c = lax.axis_index("c")
    s = lax.axis_index("s")
    is_worker = jnp.logical_and(c == 0, s < N_THR)
    lane = lax.broadcasted_iota(jnp.int32, (16,), 0)

    @pl.when(is_worker)
    def _work():
        copies = [
            pltpu.async_copy(v4h, v4, sem),
            pltpu.async_copy(predh, predv, sem),
            pltpu.async_copy(gth, gtv, sem),
            pltpu.async_copy(thrh, thrv, sem),
        ]
        for cp in copies:
            cp.wait()

        thrc = thrv[pl.ds(s * 16, 16)]
        thr = thrc[0]

        def zero_v(k, _):
            matched_v[pl.ds(k * 64, 16)] = jnp.zeros((16,), jnp.float32)
            matched_v[pl.ds(k * 64 + 16, 16)] = jnp.zeros((16,), jnp.float32)
            matched_v[pl.ds(k * 64 + 32, 16)] = jnp.zeros((16,), jnp.float32)
            matched_v[pl.ds(k * 64 + 48, 16)] = jnp.zeros((16,), jnp.float32)
            return 0

        lax.fori_loop(0, N_GT_PAD // 64, zero_v, 0)

        def zero_s(k, _):
            for u in range(8):
                matched_s[k * 8 + u] = 0
            return 0

        lax.fori_loop(0, N_GT_PAD // 8, zero_s, 0)

        def set_match(idx):
            matched_s[idx] = 1
            plsc.store_scatter(
                matched_v,
                [jnp.zeros((16,), jnp.int32) + idx],
                jnp.ones((16,), jnp.float32),
                mask=lane == 0)

        def rescan(gv):
            p0x = plsc.load_gather(predv, [gv])[0]
            p0y = plsc.load_gather(predv, [gv + 1])[0]
            p1x = plsc.load_gather(predv, [gv + 2])[0]
            p1y = plsc.load_gather(predv, [gv + 3])[0]
            parea = (p1x - p0x) * (p1y - p0y)

            def chunk(k, carry):
                bestv, besti = carry
                off = k * 16
                goff = (lane + off) * 4
                cg0x = plsc.load_gather(gtv, [goff])
                cg0y = plsc.load_gather(gtv, [goff + 1])
                cg1x = plsc.load_gather(gtv, [goff + 2])
                cg1y = plsc.load_gather(gtv, [goff + 3])
                garea = (cg1x - cg0x) * (cg1y - cg0y)
                ltx = jnp.maximum(p0x, cg0x)
                lty = jnp.maximum(p0y, cg0y)
                rbx = jnp.minimum(p1x, cg1x)
                rby = jnp.minimum(p1y, cg1y)
                whx = jnp.maximum(rbx - ltx, 0.0)
                why = jnp.maximum(rby - lty, 0.0)
                inter = whx * why
                iou = inter / (parea + garea - inter)
                mch = matched_v[pl.ds(off, 16)]
                mskd = jnp.where(mch > 0.0, -1.0, iou)
                upd = mskd > bestv
                return (jnp.where(upd, mskd, bestv),
                        jnp.where(upd, lane + off, besti))

            bestv, besti = lax.fori_loop(
                0, N_GT_PAD // 16, chunk,
                (jnp.full((16,), -2.0, jnp.float32),
                 jnp.zeros((16,), jnp.int32)))
            sv, _si = plsc.sort_key_val(bestv, besti, descending=True)
            m = sv[0]

            @pl.when(m >= thr)
            def _():
                cand = jnp.where(bestv == m, besti, 2 * N_GT_PAD)
                ci, _cv = plsc.sort_key_val(cand, cand)
                set_match(ci[0])

        def chunkrow(k, _):
            base = k * 16
            v0c = plsc.load_gather(v4, [(lane + base) * 4])
            cmask = v0c >= thr
            ncand = plsc.all_reduce_population_count(cmask)

            @pl.when(ncand[0] > 0)
            def _cands():
                def one_cand(carry):
                    mask, n = carry
                    jv = plsc.all_reduce_ffs(mask)
                    gv = (jv + base) * 4
                    idx0 = plsc.bitcast(
                        plsc.load_gather(v4, [gv + 2]), jnp.int32)[0]
                    m0 = matched_s[idx0]

                    @pl.when(m0 == 0)
                    def _():
                        set_match(idx0)

                    @pl.when(m0 != 0)
                    def _():
                        val1 = plsc.load_gather(v4, [gv + 1])[0]

                        @pl.when(val1 >= thr)
                        def _():
                            idx1 = plsc.bitcast(
                                plsc.load_gather(v4, [gv + 3]), jnp.int32)[0]
                            m1 = matched_s[idx1]

                            @pl.when(m1 == 0)
                            def _():
                                set_match(idx1)

                            @pl.when(m1 != 0)
                            def _():
                                rescan(gv)

                    return (jnp.logical_and(mask, lane != jv), n - 1)

                lax.while_loop(lambda cr: cr[1] > 0, one_cand,
                               (cmask, ncand[0]))

            return 0

        lax.fori_loop(0, N_PP // 16, chunkrow, 0)

        def acc(k, a):
            mch = matched_v[pl.ds(k * 16, 16)] > 0.0
            return a + plsc.all_reduce_population_count(mch)[0]

        tpc = lax.fori_loop(0, N_GT_PAD // 16, acc, jnp.int32(0))
        tpv = jnp.zeros((16,), jnp.float32) + tpc.astype(jnp.float32)
        outv[...] = tpv / (6000.0 - tpv)
        pltpu.sync_copy(outv, outh.at[s])

    plsc.subcore_barrier()

    @pl.when(jnp.logical_and(c == 0, s == 0))
    def _final():
        total = outv[...]
        for t in range(1, N_THR):
            pltpu.async_copy(outh.at[t], accv, sem).wait()
            total = total + accv[...]
        outv[...] = total * (1.0 / N_THR)
        pltpu.sync_copy(outv, outh.at[0])


def kernel(pred_boxes, gt_boxes):
    pad_box = jnp.tile(
        jnp.asarray([[0.0, 0.0, 1.0, 1.0]], jnp.float32), (N_PP - N_PRED, 1))
    pred_pad = jnp.concatenate([pred_boxes, pad_box], axis=0)
    gt_pad = jnp.concatenate(
        [gt_boxes, jnp.full((N_GT_PAD - N_GT, 4), _BIG, jnp.float32)], axis=0)
    tc_args = [pred_boxes]
    tc_args += [gt_pad[:, k].reshape(1, N_GT_PAD) for k in range(4)]
    v4 = pl.pallas_call(
        _tc_top2,
        out_shape=jax.ShapeDtypeStruct((N_PP, 4), jnp.float32),
    )(*tc_args)

    thr = np.full((16, 16), 2.0, np.float32)
    thr[:N_THR, :] = _THR[:, None]
    mesh = plsc.VectorSubcoreMesh(core_axis_name="c", subcore_axis_name="s")
    sck = functools.partial(
        pl.kernel,
        mesh=mesh,
        out_type=jax.ShapeDtypeStruct((8, 16), jnp.float32),
        compiler_params=pltpu.CompilerParams(needs_layout_passes=False),
        scratch_types=[
            pltpu.VMEM((N_PP * 4,), jnp.float32),
            pltpu.VMEM((N_PP * 4,), jnp.float32),
            pltpu.VMEM((N_GT_PAD * 4,), jnp.float32),
            pltpu.VMEM((16 * 16,), jnp.float32),
            pltpu.VMEM((N_GT_PAD,), jnp.float32),
            pltpu.SMEM((N_GT_PAD,), jnp.int32),
            pltpu.VMEM((16,), jnp.float32),
            pltpu.VMEM((16,), jnp.float32),
            pltpu.SemaphoreType.DMA,
        ],
    )(_sc_greedy)
    out = sck(
        v4.reshape(N_PP * 4),
        pred_pad.reshape(N_PP * 4),
        gt_pad.reshape(N_GT_PAD * 4),
        jnp.asarray(thr.reshape(16 * 16)),
    )
    return out[0, 0]
